# baseline (speedup 1.0000x reference)
import functools

import jax
import jax.numpy as jnp
from jax import lax
from jax.experimental import pallas as pl
from jax.experimental.pallas import tpu as pltpu
from jax.experimental.pallas import tpu_sc as plsc

_CTX = 200
_PATH = 20
_EMB = 64
_CTX_PAD = 256
_PATH_PAD = 32
_LN2 = 0.6931471805599453


def _plog(x):
    bits = plsc.bitcast(x, jnp.int32)
    e = ((bits >> 23) & 0xFF) - 127
    m = plsc.bitcast((bits & 0x7FFFFF) | 0x3F800000, jnp.float32)
    big = m > 1.4142135623730951
    m = jnp.where(big, m * 0.5, m)
    ef = (e + jnp.where(big, 1, 0)).astype(jnp.float32)
    t = m - 1.0
    s = t / (t + 2.0)
    z = s * s
    poly = 1.0 + z * (1.0 / 3.0 + z * (1.0 / 5.0 + z * (1.0 / 7.0 + z * (1.0 / 9.0))))
    return ef * _LN2 + 2.0 * s * poly


def _body(ctx_idx, nid, codes, valid, in_embed, node_embed, out_hbm,
          idx_v, nidx_v, rows_v, u_v, codes_v, valid_v, v_vec, out_v, sem):
    at_work = (lax.axis_index("c") == 0) & (lax.axis_index("s") == 0)

    @pl.when(at_work)
    def _():
        pltpu.sync_copy(ctx_idx, idx_v)
        pltpu.sync_copy(nid, nidx_v)
        pltpu.sync_copy(codes, codes_v)
        pltpu.sync_copy(valid, valid_v)
        cp0 = pltpu.async_copy(in_embed.at[idx_v.at[0]], rows_v.at[0], sem)
        cp1 = pltpu.async_copy(in_embed.at[idx_v.at[1]], rows_v.at[1], sem)
        cp2 = pltpu.async_copy(node_embed.at[nidx_v], u_v, sem)
        cp0.wait()
        cp1.wait()
        cp2.wait()

        zero = jnp.zeros((16,), jnp.float32)

        def chunk_sum(chunk, n, acc):
            def body(r, carry):
                a0, a1, a2, a3 = carry
                a0 = a0 + rows_v[chunk, r, pl.ds(0, 16)]
                a1 = a1 + rows_v[chunk, r, pl.ds(16, 16)]
                a2 = a2 + rows_v[chunk, r, pl.ds(32, 16)]
                a3 = a3 + rows_v[chunk, r, pl.ds(48, 16)]
                return a0, a1, a2, a3
            return lax.fori_loop(0, n, body, acc)

        acc = chunk_sum(0, 128, (zero, zero, zero, zero))
        acc = chunk_sum(1, _CTX - 128, acc)
        inv = 1.0 / _CTX
        v0, v1, v2, v3 = (a * inv for a in acc)

        v_vec[pl.ds(0, 16)] = v0
        v_vec[pl.ds(16, 16)] = v1
        v_vec[pl.ds(32, 16)] = v2
        v_vec[pl.ds(48, 16)] = v3
        lanes = lax.iota(jnp.int32, 16)

        def col_body(k, carry):
            lg0, lg1 = carry
            kk = jnp.full((16,), 0, jnp.int32) + k
            vk = plsc.load_gather(v_vec, [kk])
            col0 = plsc.load_gather(u_v, [lanes, kk])
            col1 = plsc.load_gather(u_v, [lanes + 16, kk])
            return lg0 + col0 * vk, lg1 + col1 * vk
        lg0, lg1 = lax.fori_loop(0, _EMB, col_body, (zero, zero))

        terms = zero
        for h, lg in ((0, lg0), (1, lg1)):
            cd = codes_v[pl.ds(16 * h, 16)]
            vl = valid_v[pl.ds(16 * h, 16)]
            sg = 1.0 / (1.0 + jnp.exp(-lg))
            p = jnp.where(cd == 1.0, sg, 1.0 - sg)
            terms = terms + _plog(p + 1e-9) * vl

        x = terms
        for m in (8, 4, 2, 1):
            out_v[...] = x
            x = x + plsc.load_gather(out_v, [lanes ^ m])
        out_v[...] = -x
        pltpu.sync_copy(out_v, out_hbm)


@jax.jit
def _sc_call(ctx_idx, nid, codes, valid, in_embed, node_embed):
    mesh = plsc.VectorSubcoreMesh(core_axis_name="c", subcore_axis_name="s")
    return pl.kernel(
        _body,
        out_type=jax.ShapeDtypeStruct((16,), jnp.float32),
        mesh=mesh,
        compiler_params=pltpu.CompilerParams(
            needs_layout_passes=False, use_tc_tiling_on_sc=False),
        scratch_types=[
            pltpu.VMEM((2, 128), jnp.int32),
            pltpu.VMEM((_PATH_PAD,), jnp.int32),
            pltpu.VMEM((2, 128, _EMB), jnp.float32),
            pltpu.VMEM((_PATH_PAD, _EMB), jnp.float32),
            pltpu.VMEM((_PATH_PAD,), jnp.float32),
            pltpu.VMEM((_PATH_PAD,), jnp.float32),
            pltpu.VMEM((_EMB,), jnp.float32),
            pltpu.VMEM((16,), jnp.float32),
            pltpu.SemaphoreType.DMA,
        ],
    )(ctx_idx, nid, codes, valid, in_embed, node_embed)


def kernel(context_idxs, node_ids, codes, in_embed, node_embed):
    ctx = jnp.concatenate(
        [context_idxs.astype(jnp.int32),
         jnp.zeros((_CTX_PAD - _CTX,), jnp.int32)]).reshape(2, 128)
    nid = jnp.concatenate(
        [node_ids.astype(jnp.int32),
         jnp.zeros((_PATH_PAD - _PATH,), jnp.int32)])
    cod = jnp.concatenate(
        [codes.astype(jnp.float32),
         jnp.zeros((_PATH_PAD - _PATH,), jnp.float32)])
    val = jnp.concatenate(
        [jnp.ones((_PATH,), jnp.float32),
         jnp.zeros((_PATH_PAD - _PATH,), jnp.float32)])
    out = _sc_call(ctx, nid, cod, val, in_embed, node_embed)
    return out[0]


# speedup vs baseline: 25.9352x; 25.9352x over previous
import functools

import jax
import jax.numpy as jnp
from jax import lax
from jax.experimental import pallas as pl
from jax.experimental.pallas import tpu as pltpu
from jax.experimental.pallas import tpu_sc as plsc

_VOCAB = 1000000
_CTX = 200
_PATH = 20
_EMB = 64
_CTX_PAD = 256
_PATH_PAD = 32
_LAST_BLK = (_VOCAB // 128) * 128
_LN2 = 0.6931471805599453


def _plog(x):
    bits = plsc.bitcast(x, jnp.int32)
    e = ((bits >> 23) & 0xFF) - 127
    m = plsc.bitcast((bits & 0x7FFFFF) | 0x3F800000, jnp.float32)
    big = m > 1.4142135623730951
    m = jnp.where(big, m * 0.5, m)
    ef = (e + jnp.where(big, 1, 0)).astype(jnp.float32)
    t = m - 1.0
    s = t / (t + 2.0)
    z = s * s
    poly = 1.0 + z * (1.0 / 3.0 + z * (1.0 / 5.0 + z * (1.0 / 7.0 + z * (1.0 / 9.0))))
    return ef * _LN2 + 2.0 * s * poly


def _body(ctx_idx, nid, codes, valid, tbl_t, nod_t, tail_blk, out_hbm,
          idx_v, blk, nblk, nidx_v, codes_v, valid_v, acc_v, shared, sums_v,
          out_v, sem):
    cid = lax.axis_index("c")
    sid = lax.axis_index("s")
    lanes = lax.iota(jnp.int32, 16)
    zero = jnp.zeros((16,), jnp.float32)

    @pl.when(cid == 0)
    def _():
        base = sid * 16
        pltpu.sync_copy(ctx_idx.at[pl.ds(base, 16)], idx_v)
        vec = idx_v[...]

        a0, a1, a2, a3 = zero, zero, zero, zero
        for l in range(16):
            i = vec[l]
            start = pl.multiple_of((i >> 7) << 7, 128)
            in_tail = start >= _LAST_BLK

            @pl.when(jnp.logical_not(in_tail))
            def _():
                pltpu.sync_copy(tbl_t.at[:, pl.ds(start, 128)], blk)

            @pl.when(in_tail)
            def _():
                pltpu.sync_copy(tail_blk, blk)

            off = jnp.full((16,), 0, jnp.int32) + (i & 127)
            svalid = (lanes * 0 + base + l) < _CTX
            c0 = plsc.load_gather(blk, [lanes, off])
            c1 = plsc.load_gather(blk, [lanes + 16, off])
            c2 = plsc.load_gather(blk, [lanes + 32, off])
            c3 = plsc.load_gather(blk, [lanes + 48, off])
            a0 = a0 + jnp.where(svalid, c0, 0.0)
            a1 = a1 + jnp.where(svalid, c1, 0.0)
            a2 = a2 + jnp.where(svalid, c2, 0.0)
            a3 = a3 + jnp.where(svalid, c3, 0.0)

        acc_v[pl.ds(0, 16)] = a0
        acc_v[pl.ds(16, 16)] = a1
        acc_v[pl.ds(32, 16)] = a2
        acc_v[pl.ds(48, 16)] = a3
        pltpu.sync_copy(acc_v, shared.at[sid])

    plsc.subcore_barrier()

    @pl.when((cid == 0) & (sid == 0))
    def _():
        pltpu.sync_copy(shared, sums_v)
        inv = 1.0 / _CTX
        v = [zero, zero, zero, zero]
        for r in range(16):
            for q in range(4):
                v[q] = v[q] + sums_v[r, pl.ds(16 * q, 16)]
        v = [x * inv for x in v]

        pltpu.sync_copy(nid, nidx_v)
        pltpu.sync_copy(codes, codes_v)
        pltpu.sync_copy(valid, valid_v)
        pltpu.sync_copy(nod_t.at[:, pl.ds(0, 128)], nblk)
        nid0 = nidx_v[pl.ds(0, 16)]
        nid1 = nidx_v[pl.ds(16, 16)]
        lg0, lg1 = zero, zero
        for d in range(_EMB):
            vd = v[d // 16][d % 16]
            dd = jnp.full((16,), d, jnp.int32)
            lg0 = lg0 + plsc.load_gather(nblk, [dd, nid0]) * vd
            lg1 = lg1 + plsc.load_gather(nblk, [dd, nid1]) * vd

        terms = zero
        for h, lg in ((0, lg0), (1, lg1)):
            cd = codes_v[pl.ds(16 * h, 16)]
            vl = valid_v[pl.ds(16 * h, 16)]
            sg = 1.0 / (1.0 + jnp.exp(-lg))
            p = jnp.where(cd == 1.0, sg, 1.0 - sg)
            terms = terms + _plog(p + 1e-9) * vl

        x = terms
        for m in (8, 4, 2, 1):
            out_v[...] = x
            x = x + plsc.load_gather(out_v, [lanes ^ m])
        out_v[...] = -x
        pltpu.sync_copy(out_v, out_hbm)


@jax.jit
def _sc_call(ctx_idx, nid, codes, valid, tbl_t, nod_t, tail_blk):
    mesh = plsc.VectorSubcoreMesh(core_axis_name="c", subcore_axis_name="s")
    return pl.kernel(
        _body,
        out_type=jax.ShapeDtypeStruct((16,), jnp.float32),
        mesh=mesh,
        compiler_params=pltpu.CompilerParams(
            needs_layout_passes=False, use_tc_tiling_on_sc=True),
        scratch_types=[
            pltpu.VMEM((16,), jnp.int32),
            pltpu.VMEM((_EMB, 128), jnp.float32),
            pltpu.VMEM((_EMB, 128), jnp.float32),
            pltpu.VMEM((_PATH_PAD,), jnp.int32),
            pltpu.VMEM((_PATH_PAD,), jnp.float32),
            pltpu.VMEM((_PATH_PAD,), jnp.float32),
            pltpu.VMEM((_EMB,), jnp.float32),
            pltpu.VMEM_SHARED((16, _EMB), jnp.float32),
            pltpu.VMEM((16, _EMB), jnp.float32),
            pltpu.VMEM((16,), jnp.float32),
            pltpu.SemaphoreType.DMA,
        ],
    )(ctx_idx, nid, codes, valid, tbl_t, nod_t, tail_blk)


def kernel(context_idxs, node_ids, codes, in_embed, node_embed):
    ctx = jnp.concatenate(
        [context_idxs.astype(jnp.int32),
         jnp.zeros((_CTX_PAD - _CTX,), jnp.int32)])
    nid = jnp.concatenate(
        [node_ids.astype(jnp.int32),
         jnp.zeros((_PATH_PAD - _PATH,), jnp.int32)])
    cod = jnp.concatenate(
        [codes.astype(jnp.float32),
         jnp.zeros((_PATH_PAD - _PATH,), jnp.float32)])
    val = jnp.concatenate(
        [jnp.ones((_PATH,), jnp.float32),
         jnp.zeros((_PATH_PAD - _PATH,), jnp.float32)])
    tail = jnp.concatenate(
        [in_embed[_LAST_BLK:].T,
         jnp.zeros((_EMB, 128 - (_VOCAB - _LAST_BLK)), jnp.float32)], axis=1)
    out = _sc_call(ctx, nid, cod, val, in_embed.T, node_embed.T, tail)
    return out[0]


# speedup vs baseline: 34.1949x; 1.3185x over previous
import functools

import jax
import jax.numpy as jnp
from jax import lax
from jax.experimental import pallas as pl
from jax.experimental.pallas import tpu as pltpu
from jax.experimental.pallas import tpu_sc as plsc

_VOCAB = 1000000
_CTX = 200
_PATH = 20
_EMB = 64
_CTX_PAD = 256
_PATH_PAD = 32
_LAST_BLK = (_VOCAB // 128) * 128
_LN2 = 0.6931471805599453


def _plog(x):
    bits = plsc.bitcast(x, jnp.int32)
    e = ((bits >> 23) & 0xFF) - 127
    m = plsc.bitcast((bits & 0x7FFFFF) | 0x3F800000, jnp.float32)
    big = m > 1.4142135623730951
    m = jnp.where(big, m * 0.5, m)
    ef = (e + jnp.where(big, 1, 0)).astype(jnp.float32)
    t = m - 1.0
    s = t / (t + 2.0)
    z = s * s
    poly = 1.0 + z * (1.0 / 3.0 + z * (1.0 / 5.0 + z * (1.0 / 7.0 + z * (1.0 / 9.0))))
    return ef * _LN2 + 2.0 * s * poly


def _body(ctx_idx, nid, codes, valid, tbl_t, nod_t, tail_blk, out_hbm,
          idx_v, blk, nblk, nidx_v, codes_v, valid_v, acc_v, shared, sums_v,
          out_v, sem):
    cid = lax.axis_index("c")
    sid = lax.axis_index("s")
    lanes = lax.iota(jnp.int32, 16)
    zero = jnp.zeros((16,), jnp.float32)

    @pl.when(cid == 0)
    def _():
        base = sid * 16
        pltpu.sync_copy(ctx_idx.at[pl.ds(base, 16)], idx_v)
        vec = idx_v[...]

        def issue(l):
            i = vec[l]
            start = pl.multiple_of((i >> 7) << 7, 128)
            sv = (base + l) < _CTX
            b = blk.at[l % 4]

            @pl.when(sv & (start < _LAST_BLK))
            def _():
                pltpu.async_copy(tbl_t.at[:, pl.ds(start, 128)], b, sem)

            @pl.when(sv & (start >= _LAST_BLK))
            def _():
                pltpu.async_copy(tail_blk, b, sem)

        for l in range(4):
            issue(l)

        a0, a1, a2, a3 = zero, zero, zero, zero
        for l in range(16):
            sv = (base + l) < _CTX

            @pl.when(sv)
            def _():
                pltpu.make_async_copy(tail_blk, blk.at[l % 4], sem).wait()

            off = jnp.full((16,), 0, jnp.int32) + (vec[l] & 127)
            svalid = (lanes * 0 + base + l) < _CTX
            b = blk.at[l % 4]
            c0 = plsc.load_gather(b, [lanes, off])
            c1 = plsc.load_gather(b, [lanes + 16, off])
            c2 = plsc.load_gather(b, [lanes + 32, off])
            c3 = plsc.load_gather(b, [lanes + 48, off])
            a0 = a0 + jnp.where(svalid, c0, 0.0)
            a1 = a1 + jnp.where(svalid, c1, 0.0)
            a2 = a2 + jnp.where(svalid, c2, 0.0)
            a3 = a3 + jnp.where(svalid, c3, 0.0)
            if l + 4 < 16:
                issue(l + 4)

        acc_v[pl.ds(0, 16)] = a0
        acc_v[pl.ds(16, 16)] = a1
        acc_v[pl.ds(32, 16)] = a2
        acc_v[pl.ds(48, 16)] = a3
        pltpu.sync_copy(acc_v, shared.at[sid])

    plsc.subcore_barrier()

    @pl.when((cid == 0) & (sid == 0))
    def _():
        pltpu.sync_copy(shared, sums_v)
        inv = 1.0 / _CTX
        v = [zero, zero, zero, zero]
        for r in range(16):
            for q in range(4):
                v[q] = v[q] + sums_v[r, pl.ds(16 * q, 16)]
        v = [x * inv for x in v]

        pltpu.sync_copy(nid, nidx_v)
        pltpu.sync_copy(codes, codes_v)
        pltpu.sync_copy(valid, valid_v)
        pltpu.sync_copy(nod_t.at[:, pl.ds(0, 128)], nblk)
        nid0 = nidx_v[pl.ds(0, 16)]
        nid1 = nidx_v[pl.ds(16, 16)]
        lg0, lg1 = zero, zero
        for d in range(_EMB):
            vd = v[d // 16][d % 16]
            dd = jnp.full((16,), d, jnp.int32)
            lg0 = lg0 + plsc.load_gather(nblk, [dd, nid0]) * vd
            lg1 = lg1 + plsc.load_gather(nblk, [dd, nid1]) * vd

        terms = zero
        for h, lg in ((0, lg0), (1, lg1)):
            cd = codes_v[pl.ds(16 * h, 16)]
            vl = valid_v[pl.ds(16 * h, 16)]
            sg = 1.0 / (1.0 + jnp.exp(-lg))
            p = jnp.where(cd == 1.0, sg, 1.0 - sg)
            terms = terms + _plog(p + 1e-9) * vl

        x = terms
        for m in (8, 4, 2, 1):
            out_v[...] = x
            x = x + plsc.load_gather(out_v, [lanes ^ m])
        out_v[...] = -x
        pltpu.sync_copy(out_v, out_hbm)


@jax.jit
def _sc_call(ctx_idx, nid, codes, valid, tbl_t, nod_t, tail_blk):
    mesh = plsc.VectorSubcoreMesh(core_axis_name="c", subcore_axis_name="s")
    return pl.kernel(
        _body,
        out_type=jax.ShapeDtypeStruct((16,), jnp.float32),
        mesh=mesh,
        compiler_params=pltpu.CompilerParams(
            needs_layout_passes=False, use_tc_tiling_on_sc=True),
        scratch_types=[
            pltpu.VMEM((16,), jnp.int32),
            pltpu.VMEM((4, _EMB, 128), jnp.float32),
            pltpu.VMEM((_EMB, 128), jnp.float32),
            pltpu.VMEM((_PATH_PAD,), jnp.int32),
            pltpu.VMEM((_PATH_PAD,), jnp.float32),
            pltpu.VMEM((_PATH_PAD,), jnp.float32),
            pltpu.VMEM((_EMB,), jnp.float32),
            pltpu.VMEM_SHARED((16, _EMB), jnp.float32),
            pltpu.VMEM((16, _EMB), jnp.float32),
            pltpu.VMEM((16,), jnp.float32),
            pltpu.SemaphoreType.DMA,
        ],
    )(ctx_idx, nid, codes, valid, tbl_t, nod_t, tail_blk)


def kernel(context_idxs, node_ids, codes, in_embed, node_embed):
    ctx = jnp.concatenate(
        [context_idxs.astype(jnp.int32),
         jnp.zeros((_CTX_PAD - _CTX,), jnp.int32)])
    nid = jnp.concatenate(
        [node_ids.astype(jnp.int32),
         jnp.zeros((_PATH_PAD - _PATH,), jnp.int32)])
    cod = jnp.concatenate(
        [codes.astype(jnp.float32),
         jnp.zeros((_PATH_PAD - _PATH,), jnp.float32)])
    val = jnp.concatenate(
        [jnp.ones((_PATH,), jnp.float32),
         jnp.zeros((_PATH_PAD - _PATH,), jnp.float32)])
    tail = jnp.concatenate(
        [in_embed[_LAST_BLK:].T,
         jnp.zeros((_EMB, 128 - (_VOCAB - _LAST_BLK)), jnp.float32)], axis=1)
    out = _sc_call(ctx, nid, cod, val, in_embed.T, node_embed.T, tail)
    return out[0]


# speedup vs baseline: 39.2295x; 1.1472x over previous
import functools

import jax
import jax.numpy as jnp
from jax import lax
from jax.experimental import pallas as pl
from jax.experimental.pallas import tpu as pltpu
from jax.experimental.pallas import tpu_sc as plsc

_VOCAB = 1000000
_CTX = 200
_PATH = 20
_EMB = 64
_CTX_PAD = 256
_SLOTS = 13
_LAST_BLK = (_VOCAB // 128) * 128
_LN2 = 0.6931471805599453


def _plog(x):
    bits = plsc.bitcast(x, jnp.int32)
    e = ((bits >> 23) & 0xFF) - 127
    m = plsc.bitcast((bits & 0x7FFFFF) | 0x3F800000, jnp.float32)
    big = m > 1.4142135623730951
    m = jnp.where(big, m * 0.5, m)
    ef = (e + jnp.where(big, 1, 0)).astype(jnp.float32)
    t = m - 1.0
    s = t / (t + 2.0)
    z = s * s
    poly = 1.0 + z * (1.0 / 3.0 + z * (1.0 / 5.0 + z * (1.0 / 7.0 + z * (1.0 / 9.0))))
    return ef * _LN2 + 2.0 * s * poly


def _body(ctx_idx, nid, codes, tbl_t, nod_t, tail_blk, out_hbm,
          idx_v, blk, nblk, nidx_v, codes_v, acc_v, shared, sums_v,
          out_v, sem):
    cid = lax.axis_index("c")
    sid = lax.axis_index("s")
    lanes = lax.iota(jnp.int32, 16)
    zero = jnp.zeros((16,), jnp.float32)

    @pl.when(cid == 0)
    def _():
        base = sid * _SLOTS
        base8 = pl.multiple_of((base >> 3) << 3, 8)
        pltpu.sync_copy(ctx_idx.at[pl.ds(base8, 24)], idx_v)
        vec = idx_v[pl.ds(base - base8, 16)]

        def issue(l):
            i = vec[l]
            start = pl.multiple_of((i >> 7) << 7, 128)
            sv = (base + l) < _CTX
            b = blk.at[l % 4]

            @pl.when(sv & (start < _LAST_BLK))
            def _():
                pltpu.async_copy(tbl_t.at[:, pl.ds(start, 128)], b, sem)

            @pl.when(sv & (start >= _LAST_BLK))
            def _():
                pltpu.async_copy(tail_blk, b, sem)

        for l in range(4):
            issue(l)

        a0, a1, a2, a3 = zero, zero, zero, zero
        for l in range(_SLOTS):
            sv = (base + l) < _CTX

            @pl.when(sv)
            def _():
                pltpu.make_async_copy(tail_blk, blk.at[l % 4], sem).wait()

            off = jnp.full((16,), 0, jnp.int32) + (vec[l] & 127)
            svalid = (lanes * 0 + base + l) < _CTX
            b = blk.at[l % 4]
            c0 = plsc.load_gather(b, [lanes, off])
            c1 = plsc.load_gather(b, [lanes + 16, off])
            c2 = plsc.load_gather(b, [lanes + 32, off])
            c3 = plsc.load_gather(b, [lanes + 48, off])
            a0 = a0 + jnp.where(svalid, c0, 0.0)
            a1 = a1 + jnp.where(svalid, c1, 0.0)
            a2 = a2 + jnp.where(svalid, c2, 0.0)
            a3 = a3 + jnp.where(svalid, c3, 0.0)
            if l + 4 < _SLOTS:
                issue(l + 4)

        acc_v[pl.ds(0, 16)] = a0
        acc_v[pl.ds(16, 16)] = a1
        acc_v[pl.ds(32, 16)] = a2
        acc_v[pl.ds(48, 16)] = a3
        pltpu.sync_copy(acc_v, shared.at[sid])

    plsc.subcore_barrier()

    @pl.when((cid == 0) & (sid == 0))
    def _():
        pltpu.sync_copy(shared, sums_v)
        inv = 1.0 / _CTX
        v = [zero, zero, zero, zero]
        for r in range(16):
            for q in range(4):
                v[q] = v[q] + sums_v[r, pl.ds(16 * q, 16)]
        v = [x * inv for x in v]

        pltpu.sync_copy(nid, nidx_v.at[pl.ds(0, _PATH)])
        pltpu.sync_copy(codes, codes_v.at[pl.ds(0, _PATH)])
        pltpu.sync_copy(nod_t.at[:, pl.ds(0, 128)], nblk)
        nid0 = nidx_v[pl.ds(0, 16)]
        nid1 = jnp.where(lanes < _PATH - 16, nidx_v[pl.ds(16, 16)], 0)
        lg0, lg1 = zero, zero
        for d in range(_EMB):
            vd = v[d // 16][d % 16]
            dd = jnp.full((16,), d, jnp.int32)
            lg0 = lg0 + plsc.load_gather(nblk, [dd, nid0]) * vd
            lg1 = lg1 + plsc.load_gather(nblk, [dd, nid1]) * vd

        vl0 = jnp.where(lanes < 16, 1.0, 0.0)
        vl1 = jnp.where(lanes < _PATH - 16, 1.0, 0.0)
        terms = zero
        for lg, cd, vl in ((lg0, codes_v[pl.ds(0, 16)], vl0),
                           (lg1, codes_v[pl.ds(16, 16)], vl1)):
            sg = 1.0 / (1.0 + jnp.exp(-lg))
            p = jnp.where(cd == 1.0, sg, 1.0 - sg)
            terms = terms + _plog(p + 1e-9) * vl

        x = terms
        for m in (8, 4, 2, 1):
            out_v[...] = x
            x = x + plsc.load_gather(out_v, [lanes ^ m])
        out_v[...] = -x
        pltpu.sync_copy(out_v, out_hbm)


@jax.jit
def _sc_call(ctx_idx, nid, codes, tbl_t, nod_t, tail_blk):
    mesh = plsc.VectorSubcoreMesh(core_axis_name="c", subcore_axis_name="s")
    return pl.kernel(
        _body,
        out_type=jax.ShapeDtypeStruct((16,), jnp.float32),
        mesh=mesh,
        compiler_params=pltpu.CompilerParams(
            needs_layout_passes=False, use_tc_tiling_on_sc=True),
        scratch_types=[
            pltpu.VMEM((24,), jnp.int32),
            pltpu.VMEM((4, _EMB, 128), jnp.float32),
            pltpu.VMEM((_EMB, 128), jnp.float32),
            pltpu.VMEM((32,), jnp.int32),
            pltpu.VMEM((32,), jnp.float32),
            pltpu.VMEM((_EMB,), jnp.float32),
            pltpu.VMEM_SHARED((16, _EMB), jnp.float32),
            pltpu.VMEM((16, _EMB), jnp.float32),
            pltpu.VMEM((16,), jnp.float32),
            pltpu.SemaphoreType.DMA,
        ],
    )(ctx_idx, nid, codes, tbl_t, nod_t, tail_blk)


def kernel(context_idxs, node_ids, codes, in_embed, node_embed):
    ctx = jnp.concatenate(
        [context_idxs.astype(jnp.int32),
         jnp.zeros((_CTX_PAD - _CTX,), jnp.int32)])
    tail = jnp.concatenate(
        [in_embed[_LAST_BLK:].T,
         jnp.zeros((_EMB, 128 - (_VOCAB - _LAST_BLK)), jnp.float32)], axis=1)
    out = _sc_call(ctx, node_ids.astype(jnp.int32), codes.astype(jnp.float32),
                   in_embed.T, node_embed.T, tail)
    return out[0]


# speedup vs baseline: 41.1519x; 1.0490x over previous
import functools

import jax
import jax.numpy as jnp
from jax import lax
from jax.experimental import pallas as pl
from jax.experimental.pallas import tpu as pltpu
from jax.experimental.pallas import tpu_sc as plsc

_VOCAB = 1000000
_CTX = 200
_PATH = 20
_EMB = 64
_CTX_PAD = 256
_SLOTS = 13
_LAST_BLK = (_VOCAB // 128) * 128
_LN2 = 0.6931471805599453


def _plog(x):
    bits = plsc.bitcast(x, jnp.int32)
    e = ((bits >> 23) & 0xFF) - 127
    m = plsc.bitcast((bits & 0x7FFFFF) | 0x3F800000, jnp.float32)
    big = m > 1.4142135623730951
    m = jnp.where(big, m * 0.5, m)
    ef = (e + jnp.where(big, 1, 0)).astype(jnp.float32)
    t = m - 1.0
    s = t / (t + 2.0)
    z = s * s
    poly = 1.0 + z * (1.0 / 3.0 + z * (1.0 / 5.0 + z * (1.0 / 7.0 + z * (1.0 / 9.0))))
    return ef * _LN2 + 2.0 * s * poly


def _body(ctx_idx, nid, codes, tbl_t, nod_t, tail_blk, out_hbm,
          idx_v, blk, nblk, nidx_v, codes_v, acc_v, shared, sums_v,
          out_v, sem):
    sid = lax.axis_index("s")
    lanes = lax.iota(jnp.int32, 16)
    zero = jnp.zeros((16,), jnp.float32)

    def _gather_phase():
        base = sid * _SLOTS
        base8 = pl.multiple_of((base >> 3) << 3, 8)

        @pl.when(sid < 15)
        def _():
            pltpu.sync_copy(ctx_idx.at[pl.ds(base8, 24)], idx_v.at[pl.ds(0, 24)])

        @pl.when(sid == 15)
        def _():
            pltpu.sync_copy(ctx_idx.at[pl.ds(_CTX - 8, 8)], idx_v.at[pl.ds(0, 8)])

        vec = idx_v[pl.ds(base - base8, 16)]

        def issue(l):
            i = vec[l]
            start = pl.multiple_of((i >> 7) << 7, 128)
            sv = (base + l) < _CTX
            b = blk.at[l % 4]

            @pl.when(sv & (start < _LAST_BLK))
            def _():
                pltpu.async_copy(tbl_t.at[:, pl.ds(start, 128)], b, sem)

            @pl.when(sv & (start >= _LAST_BLK))
            def _():
                pltpu.async_copy(tail_blk, b, sem)

        for l in range(4):
            issue(l)

        a0, a1, a2, a3 = zero, zero, zero, zero
        for l in range(_SLOTS):
            sv = (base + l) < _CTX

            @pl.when(sv)
            def _():
                pltpu.make_async_copy(tail_blk, blk.at[l % 4], sem).wait()

            i = vec[l]
            off = (jnp.full((16,), 0, jnp.int32) + (i & 127)
                   + jnp.where(i >= _LAST_BLK, 64, 0))
            svalid = (lanes * 0 + base + l) < _CTX
            b = blk.at[l % 4]
            c0 = plsc.load_gather(b, [lanes, off])
            c1 = plsc.load_gather(b, [lanes + 16, off])
            c2 = plsc.load_gather(b, [lanes + 32, off])
            c3 = plsc.load_gather(b, [lanes + 48, off])
            a0 = a0 + jnp.where(svalid, c0, 0.0)
            a1 = a1 + jnp.where(svalid, c1, 0.0)
            a2 = a2 + jnp.where(svalid, c2, 0.0)
            a3 = a3 + jnp.where(svalid, c3, 0.0)
            if l + 4 < _SLOTS:
                issue(l + 4)

        acc_v[pl.ds(0, 16)] = a0
        acc_v[pl.ds(16, 16)] = a1
        acc_v[pl.ds(32, 16)] = a2
        acc_v[pl.ds(48, 16)] = a3
        pltpu.sync_copy(acc_v, shared.at[sid])

    _gather_phase()
    plsc.subcore_barrier()

    @pl.when(sid == 0)
    def _():
        pltpu.sync_copy(shared, sums_v)
        inv = 1.0 / _CTX
        v = [zero, zero, zero, zero]
        for r in range(16):
            for q in range(4):
                v[q] = v[q] + sums_v[r, pl.ds(16 * q, 16)]
        v = [x * inv for x in v]

        pltpu.sync_copy(nid, nidx_v.at[pl.ds(0, _PATH)])
        pltpu.sync_copy(codes, codes_v.at[pl.ds(0, _PATH)])
        pltpu.sync_copy(nod_t.at[:, pl.ds(0, 128)], nblk)
        nid0 = nidx_v[pl.ds(0, 16)]
        nid1 = jnp.where(lanes < _PATH - 16, nidx_v[pl.ds(16, 16)], 0)
        lg0, lg1 = zero, zero
        for d in range(_EMB):
            vd = v[d // 16][d % 16]
            dd = jnp.full((16,), d, jnp.int32)
            lg0 = lg0 + plsc.load_gather(nblk, [dd, nid0]) * vd
            lg1 = lg1 + plsc.load_gather(nblk, [dd, nid1]) * vd

        vl0 = jnp.where(lanes < 16, 1.0, 0.0)
        vl1 = jnp.where(lanes < _PATH - 16, 1.0, 0.0)
        terms = zero
        for lg, cd, vl in ((lg0, codes_v[pl.ds(0, 16)], vl0),
                           (lg1, codes_v[pl.ds(16, 16)], vl1)):
            sg = 1.0 / (1.0 + jnp.exp(-lg))
            p = jnp.where(cd == 1.0, sg, 1.0 - sg)
            terms = terms + _plog(p + 1e-9) * vl

        x = terms
        for m in (8, 4, 2, 1):
            out_v[...] = x
            x = x + plsc.load_gather(out_v, [lanes ^ m])
        out_v[...] = -x
        pltpu.sync_copy(out_v, out_hbm)


@jax.jit
def _sc_call(ctx_idx, nid, codes, tbl_t, nod_t, tail_blk):
    mesh = plsc.VectorSubcoreMesh(
        core_axis_name="c", subcore_axis_name="s", num_cores=1)
    return pl.kernel(
        _body,
        out_type=jax.ShapeDtypeStruct((16,), jnp.float32),
        mesh=mesh,
        compiler_params=pltpu.CompilerParams(
            needs_layout_passes=False, use_tc_tiling_on_sc=True),
        scratch_types=[
            pltpu.VMEM((24,), jnp.int32),
            pltpu.VMEM((4, _EMB, 128), jnp.float32),
            pltpu.VMEM((_EMB, 128), jnp.float32),
            pltpu.VMEM((32,), jnp.int32),
            pltpu.VMEM((32,), jnp.float32),
            pltpu.VMEM((_EMB,), jnp.float32),
            pltpu.VMEM_SHARED((16, _EMB), jnp.float32),
            pltpu.VMEM((16, _EMB), jnp.float32),
            pltpu.VMEM((16,), jnp.float32),
            pltpu.SemaphoreType.DMA,
        ],
    )(ctx_idx, nid, codes, tbl_t, nod_t, tail_blk)


def kernel(context_idxs, node_ids, codes, in_embed, node_embed):
    tail = in_embed[_VOCAB - 128:].T
    out = _sc_call(context_idxs.astype(jnp.int32), node_ids.astype(jnp.int32),
                   codes.astype(jnp.float32), in_embed.T, node_embed.T, tail)
    return out[0]


# speedup vs baseline: 42.9495x; 1.0437x over previous
import functools

import jax
import jax.numpy as jnp
from jax import lax
from jax.experimental import pallas as pl
from jax.experimental.pallas import tpu as pltpu
from jax.experimental.pallas import tpu_sc as plsc

_VOCAB = 1000000
_CTX = 200
_PATH = 20
_EMB = 64
_SLOTS = 7
_LAST_BLK = (_VOCAB // 128) * 128


def _body(ctx_idx, nid, tbl_t, nod_t, tail_blk, out_hbm,
          idx_v, blk, nblk, nidx_v, acc_v, shared, sums_v, out_v, sem, nsem):
    cid = lax.axis_index("c")
    sid = lax.axis_index("s")
    w = cid * 16 + sid
    lanes = lax.iota(jnp.int32, 16)
    zero = jnp.zeros((16,), jnp.float32)
    base = w * _SLOTS

    @pl.when(sid == 0)
    def _():
        pltpu.async_copy(nod_t.at[:, pl.ds(0, 128)], nblk, nsem)

    base8 = pl.multiple_of((base >> 3) << 3, 8)

    @pl.when(base8 + 16 <= _CTX)
    def _():
        pltpu.sync_copy(ctx_idx.at[pl.ds(base8, 16)], idx_v.at[pl.ds(0, 16)])

    @pl.when((base8 + 16 > _CTX) & (base < _CTX))
    def _():
        pltpu.sync_copy(ctx_idx.at[pl.ds(_CTX - 8, 8)], idx_v.at[pl.ds(0, 8)])

    vec = idx_v[pl.ds(base - base8, 16)]

    def issue(l):
        i = vec[l]
        start = pl.multiple_of((i >> 7) << 7, 128)
        sv = (base + l) < _CTX
        b = blk.at[l % 4]

        @pl.when(sv & (start < _LAST_BLK))
        def _():
            pltpu.async_copy(tbl_t.at[:, pl.ds(start, 128)], b, sem)

        @pl.when(sv & (start >= _LAST_BLK))
        def _():
            pltpu.async_copy(tail_blk, b, sem)

    for l in range(4):
        issue(l)

    a0, a1, a2, a3 = zero, zero, zero, zero
    for l in range(_SLOTS):
        sv = (base + l) < _CTX

        @pl.when(sv)
        def _():
            pltpu.make_async_copy(tail_blk, blk.at[l % 4], sem).wait()

        i = vec[l]
        off = (jnp.full((16,), 0, jnp.int32) + (i & 127)
               + jnp.where(i >= _LAST_BLK, 64, 0))
        svalid = (lanes * 0 + base + l) < _CTX
        b = blk.at[l % 4]
        c0 = plsc.load_gather(b, [lanes, off])
        c1 = plsc.load_gather(b, [lanes + 16, off])
        c2 = plsc.load_gather(b, [lanes + 32, off])
        c3 = plsc.load_gather(b, [lanes + 48, off])
        a0 = a0 + jnp.where(svalid, c0, 0.0)
        a1 = a1 + jnp.where(svalid, c1, 0.0)
        a2 = a2 + jnp.where(svalid, c2, 0.0)
        a3 = a3 + jnp.where(svalid, c3, 0.0)
        if l + 4 < _SLOTS:
            issue(l + 4)

    acc_v[pl.ds(0, 16)] = a0
    acc_v[pl.ds(16, 16)] = a1
    acc_v[pl.ds(32, 16)] = a2
    acc_v[pl.ds(48, 16)] = a3
    pltpu.sync_copy(acc_v, shared.at[sid])

    plsc.subcore_barrier()

    @pl.when(sid == 0)
    def _():
        pltpu.sync_copy(shared, sums_v)
        inv = 1.0 / _CTX
        v = [zero, zero, zero, zero]
        for r in range(16):
            for q in range(4):
                v[q] = v[q] + sums_v[r, pl.ds(16 * q, 16)]
        v = [x * inv for x in v]

        pltpu.sync_copy(nid, nidx_v.at[pl.ds(0, _PATH)])
        pltpu.make_async_copy(nod_t.at[:, pl.ds(0, 128)], nblk, nsem).wait()
        nid0 = nidx_v[pl.ds(0, 16)]
        nid1 = jnp.where(lanes < _PATH - 16, nidx_v[pl.ds(16, 16)], 0)
        lg0, lg1 = zero, zero
        for d in range(_EMB):
            vd = v[d // 16][d % 16]
            dd = jnp.full((16,), d, jnp.int32)
            lg0 = lg0 + plsc.load_gather(nblk, [dd, nid0]) * vd
            lg1 = lg1 + plsc.load_gather(nblk, [dd, nid1]) * vd

        out_v[pl.ds(0, 16)] = lg0
        out_v[pl.ds(16, 16)] = lg1
        pltpu.sync_copy(out_v, out_hbm.at[pl.ds(cid * 32, 32)])


def _tc_body(lg_ref, codes_ref, out_ref):
    lg = lg_ref[pl.ds(0, 32)] + lg_ref[pl.ds(32, 32)]
    cd = jnp.concatenate([codes_ref[...], jnp.zeros((12,), jnp.float32)])
    valid = (lax.iota(jnp.int32, 32) < _PATH).astype(jnp.float32)
    sg = 1.0 / (1.0 + jnp.exp(-lg))
    p = jnp.where(cd == 1.0, sg, 1.0 - sg)
    loss = -jnp.sum(jnp.log(p + 1e-9) * valid)
    out_ref[...] = jnp.full((1,), loss, jnp.float32)


@jax.jit
def _run(ctx_idx, nid, codes, tbl_t, nod_t, tail_blk):
    mesh = plsc.VectorSubcoreMesh(core_axis_name="c", subcore_axis_name="s")
    lg = pl.kernel(
        _body,
        out_type=jax.ShapeDtypeStruct((64,), jnp.float32),
        mesh=mesh,
        compiler_params=pltpu.CompilerParams(
            needs_layout_passes=False, use_tc_tiling_on_sc=True),
        scratch_types=[
            pltpu.VMEM((24,), jnp.int32),
            pltpu.VMEM((4, _EMB, 128), jnp.float32),
            pltpu.VMEM((_EMB, 128), jnp.float32),
            pltpu.VMEM((32,), jnp.int32),
            pltpu.VMEM((_EMB,), jnp.float32),
            pltpu.VMEM_SHARED((16, _EMB), jnp.float32),
            pltpu.VMEM((16, _EMB), jnp.float32),
            pltpu.VMEM((32,), jnp.float32),
            pltpu.SemaphoreType.DMA,
            pltpu.SemaphoreType.DMA,
        ],
    )(ctx_idx, nid, tbl_t, nod_t, tail_blk)
    loss = pl.pallas_call(
        _tc_body,
        out_shape=jax.ShapeDtypeStruct((1,), jnp.float32),
    )(lg, codes)
    return loss[0]


def kernel(context_idxs, node_ids, codes, in_embed, node_embed):
    tail = in_embed[_VOCAB - 128:].T
    return _run(context_idxs.astype(jnp.int32), node_ids.astype(jnp.int32),
                codes.astype(jnp.float32), in_embed.T, node_embed.T, tail)
